# baseline (speedup 1.0000x reference)
import functools

import jax
import jax.numpy as jnp
from jax import lax
from jax.experimental import pallas as pl
from jax.experimental.pallas import tpu as pltpu
from jax.experimental.pallas import tpu_sc as plsc

N = 100000
E = 1600000
G = 64
NC = 2
NS = 16
CH = 128
KB = 4
BLK = KB * CH

EP = ((E + 4 * BLK * NC * NS - 1) // (4 * BLK * NC * NS)) * (4 * BLK * NC * NS)
EP_ALLOC = EP + 2 * KB * CH
EPT_ES = EP // (NC * NS)
EPT_CS = EP // NS

NPT = N // NS

NP_SEG = ((N + CH * NC * NS - 1) // (CH * NC * NS)) * (CH * NC * NS)
NPT_SEG = NP_SEG // (NC * NS)

NT = N + 2000

_MESH = plsc.VectorSubcoreMesh(core_axis_name="c", subcore_axis_name="s")


@functools.lru_cache(maxsize=None)
def _make_spmm(cp, channel_split):
  ept = EPT_CS if channel_split else EPT_ES
  n_pairs = ept // (2 * BLK)

  @functools.partial(
      pl.kernel,
      out_type=jax.ShapeDtypeStruct((NC, N, cp), jnp.float32),
      mesh=_MESH,
      compiler_params=pltpu.CompilerParams(use_tc_tiling_on_sc=False),
      scratch_types=[
          pltpu.VMEM((2, 2 * KB, CH), jnp.int32),
          pltpu.VMEM((2, 2 * KB, CH), jnp.int32),
          pltpu.VMEM((2, BLK, cp), jnp.float32),
          pltpu.VMEM_SHARED((N + 1, cp), jnp.float32),
          pltpu.SemaphoreType.DMA,
          pltpu.SemaphoreType.DMA,
          pltpu.SemaphoreType.DMA,
          pltpu.SemaphoreType.DMA,
          pltpu.SemaphoreType.DMA,
      ],
  )
  def k(table, srcp, dstp, zeros, out, sidx, didx, rows, acc,
        semi, semg0, semg1, sems0, sems1):
    c = lax.axis_index("c")
    s = lax.axis_index("s")

    nzf, nzt = NPT // BLK, NPT % BLK
    row0 = s * NPT
    pltpu.sync_copy(zeros, rows.at[0])
    for j in range(nzf):
      pltpu.sync_copy(rows.at[0], acc.at[pl.ds(row0 + j * BLK, BLK)])
    pltpu.sync_copy(rows.at[0, pl.ds(0, nzt)],
                    acc.at[pl.ds(row0 + nzf * BLK, nzt)])

    @pl.when(s == 0)
    def _():
      pltpu.sync_copy(rows.at[0, pl.ds(0, 1)], acc.at[pl.ds(N, 1)])

    plsc.subcore_barrier()

    if channel_split:
      base_row = s * (ept // CH)
      tbl = table.at[c]
    else:
      base_row = (c * NS + s) * (ept // CH)
      tbl = table

    semg = (semg0, semg1)
    sems = (sems0, sems1)

    def load_idx(pair, slot):
      r0 = base_row + pair * (2 * KB)
      pltpu.async_copy(srcp.at[pl.ds(r0, 2 * KB)], sidx.at[slot], semi)
      pltpu.async_copy(dstp.at[pl.ds(r0, 2 * KB)], didx.at[slot], semi)

    def wait_idx(slot):
      pltpu.make_async_copy(
          srcp.at[pl.ds(0, 2 * KB)], sidx.at[slot], semi).wait()
      pltpu.make_async_copy(
          dstp.at[pl.ds(0, 2 * KB)], didx.at[slot], semi).wait()

    def process_pair(slot):
      dg = {0: [], 1: []}
      for p in (0, 1):
        for j in range(KB):
          dg[p].append(pltpu.async_copy(
              tbl.at[sidx.at[slot, p * KB + j]],
              rows.at[p, pl.ds(j * CH, CH)], semg[p]))
      ds_ = {0: [], 1: []}
      for p in (0, 1):
        for d in dg[p]:
          d.wait()
        for j in range(KB):
          ds_[p].append(pltpu.async_copy(
              rows.at[p, pl.ds(j * CH, CH)],
              acc.at[didx.at[slot, p * KB + j]], sems[p], add=True))
      for p in (0, 1):
        for d in ds_[p]:
          d.wait()

    load_idx(0, 0)

    def eloop(g, carry):
      wait_idx(0)
      load_idx(2 * g + 1, 1)
      process_pair(0)
      wait_idx(1)
      load_idx(2 * g + 2, 0)
      process_pair(1)
      return carry

    lax.fori_loop(0, n_pairs // 2, eloop, 0)
    wait_idx(0)
    plsc.subcore_barrier()

    for j in range(nzf):
      pltpu.sync_copy(acc.at[pl.ds(row0 + j * BLK, BLK)], rows.at[0])
      pltpu.sync_copy(rows.at[0], out.at[c].at[pl.ds(row0 + j * BLK, BLK)])
    rt = row0 + nzf * BLK
    pltpu.sync_copy(acc.at[pl.ds(rt, nzt)], rows.at[1, pl.ds(0, nzt)])
    pltpu.sync_copy(rows.at[1, pl.ds(0, nzt)], out.at[c].at[pl.ds(rt, nzt)])

  return k


@functools.lru_cache(maxsize=None)
def _make_segsum():
  n_chunks = NPT_SEG // CH

  @functools.partial(
      pl.kernel,
      out_type=jax.ShapeDtypeStruct((NC, G, 64), jnp.float32),
      mesh=_MESH,
      compiler_params=pltpu.CompilerParams(use_tc_tiling_on_sc=False),
      scratch_types=[
          pltpu.VMEM((CH,), jnp.int32),
          pltpu.VMEM((CH, 64), jnp.float32),
          pltpu.VMEM_SHARED((G + 1, 64), jnp.float32),
          pltpu.SemaphoreType.DMA,
      ],
  )
  def k(h, bidx, zeros, out, didx, rows, acc, sem):
    c = lax.axis_index("c")
    s = lax.axis_index("s")

    @pl.when(s == 0)
    def _():
      pltpu.sync_copy(zeros, rows)
      pltpu.sync_copy(rows.at[pl.ds(0, G + 1)], acc)

    plsc.subcore_barrier()

    base = (c * NS + s) * NPT_SEG

    def eloop(g, carry):
      off = base + g * CH
      pltpu.sync_copy(bidx.at[pl.ds(off, CH)], didx)
      pltpu.sync_copy(h.at[pl.ds(off, CH)], rows)
      pltpu.sync_copy(rows, acc.at[didx], add=True)
      return carry

    lax.fori_loop(0, n_chunks, eloop, 0)
    plsc.subcore_barrier()

    @pl.when(s == 0)
    def _():
      pltpu.sync_copy(acc.at[pl.ds(0, G)], rows.at[pl.ds(0, G)])
      pltpu.sync_copy(rows.at[pl.ds(0, G)], out.at[c])

  return k


BN = 2000
GRID = N // BN


def _full(shape):
  return pl.BlockSpec(shape, lambda i: tuple(0 for _ in shape))


def _rows(shape, axis=0):
  def imap(i):
    return tuple(i if a == axis else 0 for a in range(len(shape)))
  return pl.BlockSpec(shape, imap)


def _dis_of(deg):
  safe = jnp.where(deg > 0, deg, 1.0)
  return jnp.where(deg > 0, 1.0 / jnp.sqrt(safe), 0.0)


def _prelu(v):
  return jnp.where(v >= 0, v, 0.2 * v)


def _stage_z(degp, x8):
  def body(degp_r, x8_r, dis_o, xs_o):
    deg = degp_r[0, :, 0:1] + degp_r[1, :, 0:1]
    dis = _dis_of(deg)
    dis_o[...] = dis
    xs_o[...] = dis * x8_r[...]

  return pl.pallas_call(
      body,
      grid=(GRID,),
      in_specs=[_rows((NC, BN, 8), 1), _rows((BN, 8))],
      out_specs=[_rows((BN, 1)), _rows((BN, 8))],
      out_shape=[
          jax.ShapeDtypeStruct((N, 1), jnp.float32),
          jax.ShapeDtypeStruct((NT, 8), jnp.float32),
      ],
  )(degp, x8)


def _stage_b(s1p, dis, cp, channel_split):
  if channel_split:
    def body(s1_r, dis_r, xs_o):
      d2 = dis_r[...] * dis_r[...]
      xs_o[0, :, :] = -d2 * s1_r[0, :, :]
      xs_o[1, :, :] = -d2 * s1_r[1, :, :]
    out_spec = _rows((NC, BN, cp), 1)
    out_shape = jax.ShapeDtypeStruct((NC, NT, cp), jnp.float32)
  else:
    def body(s1_r, dis_r, xs_o):
      d2 = dis_r[...] * dis_r[...]
      xs_o[...] = -d2 * (s1_r[0, :, :] + s1_r[1, :, :])
    out_spec = _rows((BN, cp))
    out_shape = jax.ShapeDtypeStruct((NT, cp), jnp.float32)

  return pl.pallas_call(
      body,
      grid=(GRID,),
      in_specs=[_rows((NC, BN, cp), 1), _rows((BN, 1))],
      out_specs=[out_spec],
      out_shape=[out_shape],
  )(s1p, dis)[0]


def _stage_c(t, s1p, s2p, dis, W, b, cin, cout, cp, act,
             split_in, xs_mode):
  def body(*refs):
    if xs_mode is None:
      t_r, s1_r, s2_r, dis_r, w_r, b_r, h_o = refs
      xs_o = None
    else:
      t_r, s1_r, s2_r, dis_r, w_r, b_r, h_o, xs_o = refs
    if split_in:
      s1 = jnp.concatenate([s1_r[0, :, :], s1_r[1, :, :]], axis=1)
      s2 = jnp.concatenate([s2_r[0, :, :], s2_r[1, :, :]], axis=1)
    else:
      s1 = (s1_r[0, :, :] + s1_r[1, :, :])[:, :cin]
      s2 = (s2_r[0, :, :] + s2_r[1, :, :])[:, :cin]
    d = dis_r[...]
    tt = t_r[...]
    tx1 = -d * s1
    tx2 = -2.0 * d * s2 - tt
    pre = (jnp.dot(tt, w_r[0], preferred_element_type=jnp.float32)
           + jnp.dot(tx1, w_r[1], preferred_element_type=jnp.float32)
           + jnp.dot(tx2, w_r[2], preferred_element_type=jnp.float32)
           + b_r[...])
    h = act(pre)
    h_o[...] = h
    if xs_mode == "flat":
      xs_o[...] = d * h
    elif xs_mode == "split":
      xs = d * h
      half = cout // 2
      xs_o[0, :, :] = xs[:, :half]
      xs_o[1, :, :] = xs[:, half:]

  in_specs = [
      _rows((BN, cin)),
      _rows((NC, BN, cp), 1),
      _rows((NC, BN, cp), 1),
      _rows((BN, 1)),
      _full((3, cin, cout)),
      _full((1, cout)),
  ]
  h_rows = NP_SEG if xs_mode is None else N
  out_specs = [_rows((BN, cout))]
  out_shape = [jax.ShapeDtypeStruct((h_rows, cout), jnp.float32)]
  if xs_mode == "flat":
    out_specs.append(_rows((BN, cout)))
    out_shape.append(jax.ShapeDtypeStruct((NT, cout), jnp.float32))
  elif xs_mode == "split":
    out_specs.append(_rows((NC, BN, cout // 2), 1))
    out_shape.append(jax.ShapeDtypeStruct((NC, NT, cout // 2), jnp.float32))

  return pl.pallas_call(
      body,
      grid=(GRID,),
      in_specs=in_specs,
      out_specs=out_specs,
      out_shape=out_shape,
  )(t, s1p, s2p, dis, W, b.reshape(1, cout))


def kernel(x, edge_index, batch, W1, b1, W2, b2, W3, b3):
  src = jnp.concatenate(
      [edge_index[0].astype(jnp.int32),
       jnp.full((EP_ALLOC - E,), N, dtype=jnp.int32)]).reshape(EP_ALLOC // CH, CH)
  dst = jnp.concatenate(
      [edge_index[1].astype(jnp.int32),
       jnp.full((EP_ALLOC - E,), N, dtype=jnp.int32)]).reshape(EP_ALLOC // CH, CH)
  bidx = jnp.concatenate(
      [batch.astype(jnp.int32),
       jnp.full((NP_SEG - N,), G, dtype=jnp.int32)])

  z8 = jnp.zeros((BLK, 8), jnp.float32)
  z16 = jnp.zeros((BLK, 16), jnp.float32)
  z64 = jnp.zeros((CH, 64), jnp.float32)

  spmm8 = _make_spmm(8, False)
  spmm16 = _make_spmm(16, False)
  spmm_cs = _make_spmm(16, True)
  segsum = _make_segsum()

  ones8 = jnp.ones((NT, 8), jnp.float32)
  degp = spmm8(ones8, dst, src, z8)

  x8 = jnp.pad(x, ((0, 0), (0, 5)))
  dis, xs1 = _stage_z(degp, x8)

  s1p = spmm8(xs1, src, dst, z8)
  xs2 = _stage_b(s1p, dis, 8, False)
  s2p = spmm8(xs2, src, dst, z8)
  h1, xsn = _stage_c(x, s1p, s2p, dis, W1, b1, 3, 16, 8, _prelu,
                     False, "flat")

  s1p = spmm16(xsn, src, dst, z16)
  xs2 = _stage_b(s1p, dis, 16, False)
  s2p = spmm16(xs2, src, dst, z16)
  h2, xsn = _stage_c(h1, s1p, s2p, dis, W2, b2, 16, 32, 16, _prelu,
                     False, "split")

  s1p = spmm_cs(xsn, src, dst, z16)
  xs2 = _stage_b(s1p, dis, 16, True)
  s2p = spmm_cs(xs2, src, dst, z16)
  (h3,) = _stage_c(h2, s1p, s2p, dis, W3, b3, 32, 64, 16, jnp.tanh,
                   True, None)

  segp = segsum(h3, bidx, z64)
  return segp[0] + segp[1]


# speedup vs baseline: 1.3574x; 1.3574x over previous
import functools

import jax
import jax.numpy as jnp
from jax import lax
from jax.experimental import pallas as pl
from jax.experimental.pallas import tpu as pltpu
from jax.experimental.pallas import tpu_sc as plsc

N = 100000
E = 1600000
G = 64
NC = 2
NS = 16
CH = 128
KB = 4
BLK = KB * CH

EP = ((E + 2 * BLK * NC * NS - 1) // (2 * BLK * NC * NS)) * (2 * BLK * NC * NS)
EP_ALLOC = EP
EPT_ES = EP // (NC * NS)
EPT_CS = EP // NS

NPT = N // NS

NP_SEG = ((N + CH * NC * NS - 1) // (CH * NC * NS)) * (CH * NC * NS)
NPT_SEG = NP_SEG // (NC * NS)

NT = N + 2000

_MESH = plsc.VectorSubcoreMesh(core_axis_name="c", subcore_axis_name="s")


@functools.lru_cache(maxsize=None)
def _make_spmm(cp, channel_split):
  ept = EPT_CS if channel_split else EPT_ES
  n_pairs = ept // (2 * BLK)

  @functools.partial(
      pl.kernel,
      out_type=jax.ShapeDtypeStruct((NC, N, cp), jnp.float32),
      mesh=_MESH,
      compiler_params=pltpu.CompilerParams(use_tc_tiling_on_sc=False),
      scratch_types=[
          pltpu.VMEM((2 * KB, CH), jnp.int32),
          pltpu.VMEM((2 * KB, CH), jnp.int32),
          pltpu.VMEM((2, BLK, cp), jnp.float32),
          pltpu.VMEM_SHARED((N + 1, cp), jnp.float32),
          pltpu.SemaphoreType.DMA,
          pltpu.SemaphoreType.DMA,
          pltpu.SemaphoreType.DMA,
          pltpu.SemaphoreType.DMA,
          pltpu.SemaphoreType.DMA,
      ],
  )
  def k(table, srcp, dstp, zeros, out, sidx, didx, rows, acc,
        semi, semg0, semg1, sems0, sems1):
    c = lax.axis_index("c")
    s = lax.axis_index("s")

    nzf, nzt = NPT // BLK, NPT % BLK
    row0 = s * NPT
    pltpu.sync_copy(zeros, rows.at[0])
    for j in range(nzf):
      pltpu.sync_copy(rows.at[0], acc.at[pl.ds(row0 + j * BLK, BLK)])
    pltpu.sync_copy(rows.at[0, pl.ds(0, nzt)],
                    acc.at[pl.ds(row0 + nzf * BLK, nzt)])

    @pl.when(s == 0)
    def _():
      pltpu.sync_copy(rows.at[0, pl.ds(0, 1)], acc.at[pl.ds(N, 1)])

    plsc.subcore_barrier()

    if channel_split:
      base_row = s * (ept // CH)
      tbl = table.at[c]
    else:
      base_row = (c * NS + s) * (ept // CH)
      tbl = table

    semg = (semg0, semg1)
    sems = (sems0, sems1)

    def eloop(g, carry):
      r0 = base_row + g * (2 * KB)
      di = []
      di.append(pltpu.async_copy(srcp.at[pl.ds(r0, 2 * KB)], sidx, semi))
      di.append(pltpu.async_copy(dstp.at[pl.ds(r0, 2 * KB)], didx, semi))
      for d in di:
        d.wait()
      dg = {0: [], 1: []}
      for p in (0, 1):
        for j in range(KB):
          dg[p].append(pltpu.async_copy(
              tbl.at[sidx.at[p * KB + j]],
              rows.at[p, pl.ds(j * CH, CH)], semg[p]))
      ds_ = {0: [], 1: []}
      for p in (0, 1):
        for d in dg[p]:
          d.wait()
        for j in range(KB):
          ds_[p].append(pltpu.async_copy(
              rows.at[p, pl.ds(j * CH, CH)],
              acc.at[didx.at[p * KB + j]], sems[p], add=True))
      for p in (0, 1):
        for d in ds_[p]:
          d.wait()
      return carry

    lax.fori_loop(0, n_pairs, eloop, 0)
    plsc.subcore_barrier()

    for j in range(nzf):
      pltpu.sync_copy(acc.at[pl.ds(row0 + j * BLK, BLK)], rows.at[0])
      pltpu.sync_copy(rows.at[0], out.at[c].at[pl.ds(row0 + j * BLK, BLK)])
    rt = row0 + nzf * BLK
    pltpu.sync_copy(acc.at[pl.ds(rt, nzt)], rows.at[1, pl.ds(0, nzt)])
    pltpu.sync_copy(rows.at[1, pl.ds(0, nzt)], out.at[c].at[pl.ds(rt, nzt)])

  return k


@functools.lru_cache(maxsize=None)
def _make_segsum():
  n_chunks = NPT_SEG // CH

  @functools.partial(
      pl.kernel,
      out_type=jax.ShapeDtypeStruct((NC, G, 64), jnp.float32),
      mesh=_MESH,
      compiler_params=pltpu.CompilerParams(use_tc_tiling_on_sc=False),
      scratch_types=[
          pltpu.VMEM((CH,), jnp.int32),
          pltpu.VMEM((CH, 64), jnp.float32),
          pltpu.VMEM_SHARED((G + 1, 64), jnp.float32),
          pltpu.SemaphoreType.DMA,
      ],
  )
  def k(h, bidx, zeros, out, didx, rows, acc, sem):
    c = lax.axis_index("c")
    s = lax.axis_index("s")

    @pl.when(s == 0)
    def _():
      pltpu.sync_copy(zeros, rows)
      pltpu.sync_copy(rows.at[pl.ds(0, G + 1)], acc)

    plsc.subcore_barrier()

    base = (c * NS + s) * NPT_SEG

    def eloop(g, carry):
      off = base + g * CH
      pltpu.sync_copy(bidx.at[pl.ds(off, CH)], didx)
      pltpu.sync_copy(h.at[pl.ds(off, CH)], rows)
      pltpu.sync_copy(rows, acc.at[didx], add=True)
      return carry

    lax.fori_loop(0, n_chunks, eloop, 0)
    plsc.subcore_barrier()

    @pl.when(s == 0)
    def _():
      pltpu.sync_copy(acc.at[pl.ds(0, G)], rows.at[pl.ds(0, G)])
      pltpu.sync_copy(rows.at[pl.ds(0, G)], out.at[c])

  return k


BN = 2000
GRID = N // BN


def _full(shape):
  return pl.BlockSpec(shape, lambda i: tuple(0 for _ in shape))


def _rows(shape, axis=0):
  def imap(i):
    return tuple(i if a == axis else 0 for a in range(len(shape)))
  return pl.BlockSpec(shape, imap)


def _dis_of(deg):
  safe = jnp.where(deg > 0, deg, 1.0)
  return jnp.where(deg > 0, 1.0 / jnp.sqrt(safe), 0.0)


def _prelu(v):
  return jnp.where(v >= 0, v, 0.2 * v)


def _stage_z(degp, x8):
  def body(degp_r, x8_r, dis_o, xs_o):
    deg = degp_r[0, :, 0:1] + degp_r[1, :, 0:1]
    dis = _dis_of(deg)
    dis_o[...] = dis
    xs_o[...] = dis * x8_r[...]

  return pl.pallas_call(
      body,
      grid=(GRID,),
      in_specs=[_rows((NC, BN, 8), 1), _rows((BN, 8))],
      out_specs=[_rows((BN, 1)), _rows((BN, 8))],
      out_shape=[
          jax.ShapeDtypeStruct((N, 1), jnp.float32),
          jax.ShapeDtypeStruct((NT, 8), jnp.float32),
      ],
  )(degp, x8)


def _stage_b(s1p, dis, cp, channel_split):
  if channel_split:
    def body(s1_r, dis_r, xs_o):
      d2 = dis_r[...] * dis_r[...]
      xs_o[0, :, :] = -d2 * s1_r[0, :, :]
      xs_o[1, :, :] = -d2 * s1_r[1, :, :]
    out_spec = _rows((NC, BN, cp), 1)
    out_shape = jax.ShapeDtypeStruct((NC, NT, cp), jnp.float32)
  else:
    def body(s1_r, dis_r, xs_o):
      d2 = dis_r[...] * dis_r[...]
      xs_o[...] = -d2 * (s1_r[0, :, :] + s1_r[1, :, :])
    out_spec = _rows((BN, cp))
    out_shape = jax.ShapeDtypeStruct((NT, cp), jnp.float32)

  return pl.pallas_call(
      body,
      grid=(GRID,),
      in_specs=[_rows((NC, BN, cp), 1), _rows((BN, 1))],
      out_specs=[out_spec],
      out_shape=[out_shape],
  )(s1p, dis)[0]


def _stage_c(t, s1p, s2p, dis, W, b, cin, cout, cp, act,
             split_in, xs_mode):
  def body(*refs):
    if xs_mode is None:
      t_r, s1_r, s2_r, dis_r, w_r, b_r, h_o = refs
      xs_o = None
    else:
      t_r, s1_r, s2_r, dis_r, w_r, b_r, h_o, xs_o = refs
    if split_in:
      s1 = jnp.concatenate([s1_r[0, :, :], s1_r[1, :, :]], axis=1)
      s2 = jnp.concatenate([s2_r[0, :, :], s2_r[1, :, :]], axis=1)
    else:
      s1 = (s1_r[0, :, :] + s1_r[1, :, :])[:, :cin]
      s2 = (s2_r[0, :, :] + s2_r[1, :, :])[:, :cin]
    d = dis_r[...]
    tt = t_r[...]
    tx1 = -d * s1
    tx2 = -2.0 * d * s2 - tt
    pre = (jnp.dot(tt, w_r[0], preferred_element_type=jnp.float32)
           + jnp.dot(tx1, w_r[1], preferred_element_type=jnp.float32)
           + jnp.dot(tx2, w_r[2], preferred_element_type=jnp.float32)
           + b_r[...])
    h = act(pre)
    h_o[...] = h
    if xs_mode == "flat":
      xs_o[...] = d * h
    elif xs_mode == "split":
      xs = d * h
      half = cout // 2
      xs_o[0, :, :] = xs[:, :half]
      xs_o[1, :, :] = xs[:, half:]

  in_specs = [
      _rows((BN, cin)),
      _rows((NC, BN, cp), 1),
      _rows((NC, BN, cp), 1),
      _rows((BN, 1)),
      _full((3, cin, cout)),
      _full((1, cout)),
  ]
  h_rows = NP_SEG if xs_mode is None else N
  out_specs = [_rows((BN, cout))]
  out_shape = [jax.ShapeDtypeStruct((h_rows, cout), jnp.float32)]
  if xs_mode == "flat":
    out_specs.append(_rows((BN, cout)))
    out_shape.append(jax.ShapeDtypeStruct((NT, cout), jnp.float32))
  elif xs_mode == "split":
    out_specs.append(_rows((NC, BN, cout // 2), 1))
    out_shape.append(jax.ShapeDtypeStruct((NC, NT, cout // 2), jnp.float32))

  return pl.pallas_call(
      body,
      grid=(GRID,),
      in_specs=in_specs,
      out_specs=out_specs,
      out_shape=out_shape,
  )(t, s1p, s2p, dis, W, b.reshape(1, cout))


def kernel(x, edge_index, batch, W1, b1, W2, b2, W3, b3):
  src = jnp.concatenate(
      [edge_index[0].astype(jnp.int32),
       jnp.full((EP_ALLOC - E,), N, dtype=jnp.int32)]).reshape(EP_ALLOC // CH, CH)
  dst = jnp.concatenate(
      [edge_index[1].astype(jnp.int32),
       jnp.full((EP_ALLOC - E,), N, dtype=jnp.int32)]).reshape(EP_ALLOC // CH, CH)
  bidx = jnp.concatenate(
      [batch.astype(jnp.int32),
       jnp.full((NP_SEG - N,), G, dtype=jnp.int32)])

  z8 = jnp.zeros((BLK, 8), jnp.float32)
  z16 = jnp.zeros((BLK, 16), jnp.float32)
  z64 = jnp.zeros((CH, 64), jnp.float32)

  spmm8 = _make_spmm(8, False)
  spmm16 = _make_spmm(16, False)
  spmm_cs = _make_spmm(16, True)
  segsum = _make_segsum()

  ones8 = jnp.ones((NT, 8), jnp.float32)
  degp = spmm8(ones8, dst, src, z8)

  x8 = jnp.pad(x, ((0, 0), (0, 5)))
  dis, xs1 = _stage_z(degp, x8)

  s1p = spmm8(xs1, src, dst, z8)
  xs2 = _stage_b(s1p, dis, 8, False)
  s2p = spmm8(xs2, src, dst, z8)
  h1, xsn = _stage_c(x, s1p, s2p, dis, W1, b1, 3, 16, 8, _prelu,
                     False, "flat")

  s1p = spmm16(xsn, src, dst, z16)
  xs2 = _stage_b(s1p, dis, 16, False)
  s2p = spmm16(xs2, src, dst, z16)
  h2, xsn = _stage_c(h1, s1p, s2p, dis, W2, b2, 16, 32, 16, _prelu,
                     False, "split")

  s1p = spmm_cs(xsn, src, dst, z16)
  xs2 = _stage_b(s1p, dis, 16, True)
  s2p = spmm_cs(xs2, src, dst, z16)
  (h3,) = _stage_c(h2, s1p, s2p, dis, W3, b3, 32, 64, 16, jnp.tanh,
                   True, None)

  segp = segsum(h3, bidx, z64)
  return segp[0] + segp[1]
